# baseline (device time: 20804 ns/iter reference)
import jax
import jax.numpy as jnp
from jax import lax
from jax.experimental import pallas as pl
from jax.experimental.pallas import tpu as pltpu

N_DEV = 8
EPS = 1e-5


def kernel(x, t_emb, W_scale, W_shift):
    b, s, c_loc = x.shape
    c_global = c_loc * N_DEV

    def body(x_ref, t_ref, wsc_ref, wsh_ref, out_ref,
             comm_ref, send_sems, recv_sems):
        my = lax.axis_index("i")

        xs = x_ref[...]
        comm_ref[N_DEV - 1, 0] = jnp.sum(xs, axis=-1)
        comm_ref[N_DEV - 1, 1] = jnp.sum(xs * xs, axis=-1)

        barrier = pltpu.get_barrier_semaphore()
        for d in range(1, N_DEV):
            tgt = lax.rem(my + d, N_DEV)
            pl.semaphore_signal(
                barrier, inc=1,
                device_id=(tgt,), device_id_type=pl.DeviceIdType.MESH,
            )
        pl.semaphore_wait(barrier, N_DEV - 1)

        rdmas = []
        for d in range(1, N_DEV):
            tgt = lax.rem(my + d, N_DEV)
            rdma = pltpu.make_async_remote_copy(
                src_ref=comm_ref.at[N_DEV - 1],
                dst_ref=comm_ref.at[d - 1],
                send_sem=send_sems.at[d - 1],
                recv_sem=recv_sems.at[d - 1],
                device_id=(tgt,),
                device_id_type=pl.DeviceIdType.MESH,
            )
            rdma.start()
            rdmas.append(rdma)

        t = t_ref[...]
        scale = jnp.dot(t, wsc_ref[...], preferred_element_type=jnp.float32)
        shift = jnp.dot(t, wsh_ref[...], preferred_element_type=jnp.float32)

        for rdma in rdmas:
            rdma.wait_recv()

        tot = jnp.sum(comm_ref[...], axis=0)
        mean = tot[0] / c_global
        var = tot[1] / c_global - mean * mean
        inv = lax.rsqrt(var + EPS)

        h = (xs - mean[:, :, None]) * inv[:, :, None]
        out_ref[...] = h * (1.0 + scale)[:, None, :] + shift[:, None, :]

        for rdma in rdmas:
            rdma.wait_send()

    return pl.pallas_call(
        body,
        out_shape=jax.ShapeDtypeStruct((b, s, c_loc), jnp.float32),
        in_specs=[pl.BlockSpec(memory_space=pltpu.VMEM)] * 4,
        out_specs=pl.BlockSpec(memory_space=pltpu.VMEM),
        scratch_shapes=[
            pltpu.VMEM((N_DEV, 2, b, s), jnp.float32),
            pltpu.SemaphoreType.DMA((N_DEV - 1,)),
            pltpu.SemaphoreType.DMA((N_DEV - 1,)),
        ],
        compiler_params=pltpu.CompilerParams(collective_id=0),
    )(x, t_emb, W_scale, W_shift)
